# baseline (speedup 1.0000x reference)
import jax
import jax.numpy as jnp
from jax.experimental import pallas as pl

B, N, C = 4, 20000, 21
NP = 20480
MAX_TOTAL = 200
SCORE_THR = 0.5
IOU_THR = 0.5
NEG = jnp.float32(-1e30)


def _decode_body(roi_ref, d0_ref, d1_ref, d2_ref, d3_ref, prob_ref,
                 y1_ref, x1_ref, y2_ref, x2_ref, sc_ref):
    rois = roi_ref[0]
    probs = prob_ref[0]
    d0 = d0_ref[0] * 0.1
    d1 = d1_ref[0] * 0.1
    d2 = d2_ref[0] * 0.2
    d3 = d3_ref[0] * 0.2
    aw = (rois[:, 3] - rois[:, 1])[:, None]
    ah = (rois[:, 2] - rois[:, 0])[:, None]
    acx = rois[:, 1][:, None] + 0.5 * aw
    acy = rois[:, 0][:, None] + 0.5 * ah
    bw = jnp.exp(d3) * aw
    bh = jnp.exp(d2) * ah
    bcx = d1 * aw + acx
    bcy = d0 * ah + acy
    y1 = bcy - 0.5 * bh
    x1 = bcx - 0.5 * bw
    y1_ref[0] = y1
    x1_ref[0] = x1
    y2_ref[0] = bh + y1
    x2_ref[0] = bw + x1
    m = jnp.max(probs, axis=-1, keepdims=True)
    keep = probs[:, 0:1] < m
    sc_ref[0] = jnp.where(keep, probs, 0.0)


def _decode(rois, d0, d1, d2, d3, probs):
    blk = 1024
    grid = (B, NP // blk)
    spec_c = pl.BlockSpec((1, blk, C), lambda b, i: (b, i, 0))
    return pl.pallas_call(
        _decode_body,
        grid=grid,
        in_specs=[pl.BlockSpec((1, blk, 4), lambda b, i: (b, i, 0)),
                  spec_c, spec_c, spec_c, spec_c, spec_c],
        out_specs=[spec_c, spec_c, spec_c, spec_c, spec_c],
        out_shape=[jax.ShapeDtypeStruct((B, NP, C), jnp.float32)] * 5,
    )(rois, d0, d1, d2, d3, probs)


@jax.jit
def _nms_indices(boxes, scores):
    n = boxes.shape[0]
    idx_range = jnp.arange(n)
    areas = jnp.maximum(boxes[:, 2] - boxes[:, 0], 0.0) * jnp.maximum(boxes[:, 3] - boxes[:, 1], 0.0)

    def body(i, carry):
        sc, sel_idx, sel_valid = carry
        j = jnp.argmax(sc)
        s = sc[j]
        valid = s > SCORE_THR
        sel_idx = sel_idx.at[i].set(j.astype(jnp.int32))
        sel_valid = sel_valid.at[i].set(valid)
        b = boxes[j]
        yy1 = jnp.maximum(b[0], boxes[:, 0])
        xx1 = jnp.maximum(b[1], boxes[:, 1])
        yy2 = jnp.minimum(b[2], boxes[:, 2])
        xx2 = jnp.minimum(b[3], boxes[:, 3])
        inter = jnp.maximum(yy2 - yy1, 0.0) * jnp.maximum(xx2 - xx1, 0.0)
        iou = inter / jnp.maximum(areas[j] + areas - inter, 1e-8)
        sc = jnp.where((iou > IOU_THR) | (idx_range == j), NEG, sc)
        return sc, sel_idx, sel_valid

    init = (scores, jnp.zeros((MAX_TOTAL,), jnp.int32), jnp.zeros((MAX_TOTAL,), jnp.bool_))
    _, sel_idx, sel_valid = jax.lax.fori_loop(0, MAX_TOTAL, body, init)
    return sel_idx, sel_valid


def kernel(roi_bboxes, pred_deltas, pred_label_probs):
    pad = NP - N
    rois_p = jnp.pad(roi_bboxes, ((0, 0), (0, pad), (0, 0)))
    deltas_p = jnp.pad(pred_deltas, ((0, 0), (0, pad), (0, 0))).reshape(B, NP, C, 4)
    probs_p = jnp.pad(pred_label_probs, ((0, 0), (0, pad), (0, 0)))
    y1, x1, y2, x2, sc_all = _decode(
        rois_p, deltas_p[..., 0], deltas_p[..., 1], deltas_p[..., 2],
        deltas_p[..., 3], probs_p)
    pred_bboxes = jnp.stack([y1, x1, y2, x2], axis=-1)[:, :N]
    pred_labels = sc_all[:, :N]
    out_b, out_c, out_s = [], [], []
    for b in range(B):
        cs, cb, cc = [], [], []
        for c in range(C):
            idx, valid = _nms_indices(pred_bboxes[b, :, c, :], pred_labels[b, :, c])
            cs.append(jnp.where(valid, pred_labels[b, idx, c], NEG))
            cb.append(pred_bboxes[b, idx, c, :])
            cc.append(jnp.full((MAX_TOTAL,), float(c), jnp.float32))
        cs = jnp.concatenate(cs)
        cb = jnp.concatenate(cb, axis=0)
        cc = jnp.concatenate(cc)
        top_vals, top_idx = jax.lax.top_k(cs, MAX_TOTAL)
        valid = top_vals > (NEG * 0.5)
        out_s.append(jnp.where(valid, cs[top_idx], 0.0))
        out_b.append(jnp.where(valid[:, None], jnp.clip(cb[top_idx], 0.0, 1.0), 0.0))
        out_c.append(jnp.where(valid, cc[top_idx], 0.0))
    return jnp.stack(out_b), jnp.stack(out_c), jnp.stack(out_s)


# speedup vs baseline: 51.9547x; 51.9547x over previous
import functools

import jax
import jax.numpy as jnp
from jax import lax
from jax.experimental import pallas as pl
from jax.experimental.pallas import tpu as pltpu
from jax.experimental.pallas import tpu_sc as plsc

B, N, C = 4, 20000, 21
NP = 20480
NT = B * C
MAX_TOTAL = 200
SCORE_THR = 0.5
IOU_THR = 0.5
NEG = -1e30
KCAP = 512
OUTW = 256
L = 16
NWORK = 32


def _scores_body(prob_ref, out_ref):
    probs = prob_ref[0]
    m = jnp.max(probs, axis=-1, keepdims=True)
    keep = probs[:, 0:1] < m
    sc = jnp.where(keep, probs, 0.0)
    out_ref[0] = sc.T


def _scores_tc(probs_p):
    blk = 2048
    return pl.pallas_call(
        _scores_body,
        grid=(B, NP // blk),
        in_specs=[pl.BlockSpec((1, blk, C), lambda b, i: (b, i, 0))],
        out_specs=pl.BlockSpec((1, C, blk), lambda b, i: (b, 0, i)),
        out_shape=jax.ShapeDtypeStruct((B, C, NP), jnp.float32),
    )(probs_p)


def _nms_task(task, sc_hbm, ry1_hbm, rx1_hbm, ry2_hbm, rx2_hbm,
              d0_hbm, d1_hbm, d2_hbm, d3_hbm, osc_hbm, obox_hbm,
              dense, widx, wsc, wy1, wx1, wy2, wx2, warea,
              gidx, gidx2, gy1, gx1, gy2, gx2, gd0, gd1, gd2, gd3,
              ay1, ax1, ay2, ax2, aarea, osc, obox, sem):
    b = lax.shift_right_logical(task * 3121, 16)
    c = task - b * C
    iota = lax.iota(jnp.int32, L)
    fzero = jnp.zeros((L,), jnp.float32)
    izero = jnp.zeros((L,), jnp.int32)
    fneg = jnp.full((L,), NEG, jnp.float32)

    pltpu.sync_copy(sc_hbm.at[b, c], dense)

    def init_body(i, _):
        osc[pl.ds(i * L, L)] = fneg
        return 0
    lax.fori_loop(0, OUTW // L, init_body, 0, unroll=4)

    def initb_body(i, _):
        obox[pl.ds(i * L, L)] = fzero
        return 0
    lax.fori_loop(0, OUTW * 4 // L, initb_body, 0, unroll=4)

    def cnt_body(i, acc):
        v = dense[pl.ds(i * L, L)]
        return acc + jnp.sum((v > SCORE_THR).astype(jnp.int32))
    n05 = lax.fori_loop(0, NP // L, cnt_body, jnp.int32(0), unroll=8)

    def win_pass(t_new, t_lo):
        def body(i, wp):
            v = dense[pl.ds(i * L, L)]
            m = (v > t_new) & (v <= t_lo)
            pos = wp + plsc.cumsum(m.astype(jnp.int32)) - 1
            mm = m & (pos < KCAP)
            plsc.store_scatter(widx, [pos], iota + i * L, mask=mm)
            plsc.store_scatter(wsc, [pos], v, mask=mm)
            return wp + jnp.sum(m.astype(jnp.int32))
        return lax.fori_loop(0, NP // L, body, jnp.int32(0), unroll=4)

    def outer_cond(st):
        accepted, n_rem, t_lo = st
        return (accepted < MAX_TOTAL) & (n_rem > 0)

    def outer_body(st):
        accepted0, n_rem, t_lo = st

        hi0 = jnp.minimum(t_lo, jnp.float32(1.0))
        frac = jnp.max(jnp.full((L,), 0.75 * KCAP, jnp.float32) /
                       jnp.full((L,), jnp.maximum(n_rem, 1).astype(jnp.float32)))
        t0 = jnp.where(n_rem <= KCAP, jnp.float32(SCORE_THR),
                       jnp.maximum(jnp.float32(SCORE_THR),
                                   hi0 - (hi0 - SCORE_THR) * frac))
        cnt0 = win_pass(t0, t_lo)

        def sel_cond(s):
            t, cnt, lo, hi, tries = s
            bad = (cnt > KCAP) | ((cnt == 0) & (t > SCORE_THR))
            return bad & (tries < 30)

        def sel_body(s):
            t, cnt, lo, hi, tries = s
            lo2 = jnp.where(cnt > KCAP, t, lo)
            hi2 = jnp.where(cnt == 0, t, hi)
            t2 = 0.5 * (lo2 + hi2)
            c2 = win_pass(t2, t_lo)
            return t2, c2, lo2, hi2, tries + 1

        t_new, cnt, _, _, _ = lax.while_loop(
            sel_cond, sel_body,
            (t0, cnt0, jnp.float32(SCORE_THR), hi0, jnp.int32(0)))

        def force():
            return jnp.float32(SCORE_THR), win_pass(jnp.float32(SCORE_THR), t_lo)

        t_new, cnt = lax.cond((cnt == 0) & (n_rem > 0), force,
                              lambda: (t_new, cnt))

        nwin = jnp.minimum(cnt, KCAP)
        n_rem2 = jnp.maximum(n_rem - cnt, 0)

        def pad_body(i, _):
            lane = iota + i * L
            m = lane >= nwin
            plsc.store_scatter(wsc, [lane], fneg, mask=m)
            plsc.store_scatter(widx, [lane], izero, mask=m)
            return 0
        lax.fori_loop(0, KCAP // L, pad_body, 0, unroll=4)

        def ridx_body(i, _):
            gi = plsc.load_gather(widx, [iota + i * L])
            ri = gi + b * N
            gidx[pl.ds(i * L, L)] = ri
            gidx2[pl.ds(i * L, L)] = ri * C + c
            return 0
        lax.fori_loop(0, KCAP // L, ridx_body, 0, unroll=4)

        CH = L * 8

        def chunk(k, _):
            @pl.when(k * CH < nwin)
            def _():
                sl = pl.ds(k * CH, CH)
                cps = [
                    pltpu.async_copy(ry1_hbm.at[gidx.at[sl]], gy1, sem),
                    pltpu.async_copy(rx1_hbm.at[gidx.at[sl]], gx1, sem),
                    pltpu.async_copy(ry2_hbm.at[gidx.at[sl]], gy2, sem),
                    pltpu.async_copy(rx2_hbm.at[gidx.at[sl]], gx2, sem),
                    pltpu.async_copy(d0_hbm.at[gidx2.at[sl]], gd0, sem),
                    pltpu.async_copy(d1_hbm.at[gidx2.at[sl]], gd1, sem),
                    pltpu.async_copy(d2_hbm.at[gidx2.at[sl]], gd2, sem),
                    pltpu.async_copy(d3_hbm.at[gidx2.at[sl]], gd3, sem),
                ]
                for cp in cps:
                    cp.wait()

                def dec(i, _):
                    base = k * CH + i * L
                    d0 = gd0[pl.ds(i * L, L)] * 0.1
                    d1 = gd1[pl.ds(i * L, L)] * 0.1
                    d2 = gd2[pl.ds(i * L, L)] * 0.2
                    d3 = gd3[pl.ds(i * L, L)] * 0.2
                    r0 = gy1[pl.ds(i * L, L)]
                    r1 = gx1[pl.ds(i * L, L)]
                    r2 = gy2[pl.ds(i * L, L)]
                    r3 = gx2[pl.ds(i * L, L)]
                    aw = r3 - r1
                    ah = r2 - r0
                    acx = r1 + 0.5 * aw
                    acy = r0 + 0.5 * ah
                    bw = jnp.exp(d3) * aw
                    bh = jnp.exp(d2) * ah
                    bcx = d1 * aw + acx
                    bcy = d0 * ah + acy
                    y1 = bcy - 0.5 * bh
                    x1 = bcx - 0.5 * bw
                    y2 = bh + y1
                    x2 = bw + x1
                    wy1[pl.ds(base, L)] = y1
                    wx1[pl.ds(base, L)] = x1
                    wy2[pl.ds(base, L)] = y2
                    wx2[pl.ds(base, L)] = x2
                    warea[pl.ds(base, L)] = (jnp.maximum(y2 - y1, 0.0) *
                                             jnp.maximum(x2 - x1, 0.0))
                    return 0
                lax.fori_loop(0, 8, dec, 0, unroll=2)
            return 0
        lax.fori_loop(0, KCAP // CH, chunk, 0)

        def window_max():
            def amax_body(i, s):
                bv, bp = s
                v = wsc[pl.ds(i * L, L)]
                p = iota + i * L
                upd = v > bv
                return jnp.where(upd, v, bv), jnp.where(upd, p, bp)
            bv, bp = lax.fori_loop(0, KCAP // L, amax_body,
                                   (fneg, jnp.full((L,), KCAP, jnp.int32)),
                                   unroll=4)
            m = jnp.max(bv)
            j = jnp.min(jnp.where(bv == m, bp, jnp.int32(KCAP)))
            return m, j

        def inner_cond(st3):
            accepted, m, j = st3
            return (accepted < MAX_TOTAL) & (m > jnp.float32(NEG / 2))

        def inner_body(st3):
            accepted, m, j = st3
            jj = jnp.full((L,), j, jnp.int32)
            plsc.store_scatter(wsc, [jj], fneg, mask=iota == 0)
            cy1 = plsc.load_gather(wy1, [jj])
            cx1 = plsc.load_gather(wx1, [jj])
            cy2 = plsc.load_gather(wy2, [jj])
            cx2 = plsc.load_gather(wx2, [jj])
            car = plsc.load_gather(warea, [jj])

            nav = lax.shift_right_logical(accepted + (L - 1), 4)

            def iou_body(i, rej):
                lane_ok = (iota + i * L) < accepted
                yy1 = jnp.maximum(cy1, ay1[pl.ds(i * L, L)])
                xx1 = jnp.maximum(cx1, ax1[pl.ds(i * L, L)])
                yy2 = jnp.minimum(cy2, ay2[pl.ds(i * L, L)])
                xx2 = jnp.minimum(cx2, ax2[pl.ds(i * L, L)])
                inter = jnp.maximum(yy2 - yy1, 0.0) * jnp.maximum(xx2 - xx1, 0.0)
                iou = inter / jnp.maximum(aarea[pl.ds(i * L, L)] + car - inter, 1e-8)
                hit = (iou > IOU_THR) & lane_ok
                return rej | (jnp.max(hit.astype(jnp.int32)) > 0)
            rejected = lax.fori_loop(0, nav, iou_body, jnp.bool_(False))

            @pl.when(jnp.logical_not(rejected))
            def _():
                slot = jnp.full((L,), accepted, jnp.int32)
                lane0 = iota == 0
                plsc.store_scatter(ay1, [slot], cy1, mask=lane0)
                plsc.store_scatter(ax1, [slot], cx1, mask=lane0)
                plsc.store_scatter(ay2, [slot], cy2, mask=lane0)
                plsc.store_scatter(ax2, [slot], cx2, mask=lane0)
                plsc.store_scatter(aarea, [slot], car, mask=lane0)
                plsc.store_scatter(osc, [slot], jnp.full((L,), m, jnp.float32),
                                   mask=lane0)
                slot4 = slot * 4
                plsc.store_scatter(obox, [slot4], cy1, mask=lane0)
                plsc.store_scatter(obox, [slot4 + 1], cx1, mask=lane0)
                plsc.store_scatter(obox, [slot4 + 2], cy2, mask=lane0)
                plsc.store_scatter(obox, [slot4 + 3], cx2, mask=lane0)

            accepted2 = accepted + jnp.where(rejected, 0, 1)
            m2, j2 = window_max()
            return accepted2, m2, j2

        m0, j0 = window_max()
        accepted_f, _, _ = lax.while_loop(inner_cond, inner_body,
                                          (accepted0, m0, j0))
        return accepted_f, n_rem2, t_new

    lax.while_loop(outer_cond, outer_body,
                   (jnp.int32(0), n05, jnp.float32(2.0)))

    pltpu.sync_copy(osc, osc_hbm.at[task])
    pltpu.sync_copy(obox, obox_hbm.at[task])


def _nms_sc(scores_t, planes):
    mesh = plsc.VectorSubcoreMesh(core_axis_name="c", subcore_axis_name="s")

    @functools.partial(
        pl.kernel, mesh=mesh,
        compiler_params=pltpu.CompilerParams(needs_layout_passes=False, use_tc_tiling_on_sc=False),
        out_type=[jax.ShapeDtypeStruct((NT, OUTW), jnp.float32),
                  jax.ShapeDtypeStruct((NT, OUTW * 4), jnp.float32)],
        scratch_types=[
            pltpu.VMEM((NP,), jnp.float32),
            pltpu.VMEM((KCAP,), jnp.int32),
            pltpu.VMEM((KCAP,), jnp.float32),
            pltpu.VMEM((KCAP,), jnp.float32),
            pltpu.VMEM((KCAP,), jnp.float32),
            pltpu.VMEM((KCAP,), jnp.float32),
            pltpu.VMEM((KCAP,), jnp.float32),
            pltpu.VMEM((KCAP,), jnp.float32),
            pltpu.VMEM((KCAP,), jnp.int32),
            pltpu.VMEM((KCAP,), jnp.int32),
            pltpu.VMEM((L * 8,), jnp.float32),
            pltpu.VMEM((L * 8,), jnp.float32),
            pltpu.VMEM((L * 8,), jnp.float32),
            pltpu.VMEM((L * 8,), jnp.float32),
            pltpu.VMEM((L * 8,), jnp.float32),
            pltpu.VMEM((L * 8,), jnp.float32),
            pltpu.VMEM((L * 8,), jnp.float32),
            pltpu.VMEM((L * 8,), jnp.float32),
            pltpu.VMEM((OUTW,), jnp.float32),
            pltpu.VMEM((OUTW,), jnp.float32),
            pltpu.VMEM((OUTW,), jnp.float32),
            pltpu.VMEM((OUTW,), jnp.float32),
            pltpu.VMEM((OUTW,), jnp.float32),
            pltpu.VMEM((OUTW,), jnp.float32),
            pltpu.VMEM((OUTW * 4,), jnp.float32),
            pltpu.SemaphoreType.DMA,
        ],
    )
    def k(sc_hbm, ry1, rx1, ry2, rx2, dd0, dd1, dd2, dd3,
          osc_hbm, obox_hbm, *scr):
        wid = lax.axis_index("s") * 2 + lax.axis_index("c")
        for rep in range(3):
            task = wid + rep * NWORK

            @pl.when(task < NT)
            def _():
                _nms_task(task, sc_hbm, ry1, rx1, ry2, rx2,
                          dd0, dd1, dd2, dd3, osc_hbm, obox_hbm, *scr)

    return k(scores_t, *planes)


def _merge_sc(osc, obox):
    mesh = plsc.VectorSubcoreMesh(core_axis_name="c", subcore_axis_name="s")

    @functools.partial(
        pl.kernel, mesh=mesh,
        compiler_params=pltpu.CompilerParams(needs_layout_passes=False, use_tc_tiling_on_sc=False),
        out_type=[jax.ShapeDtypeStruct((B, MAX_TOTAL * 4), jnp.float32),
                  jax.ShapeDtypeStruct((B, MAX_TOTAL), jnp.float32),
                  jax.ShapeDtypeStruct((B, MAX_TOTAL), jnp.float32)],
        scratch_types=[
            pltpu.VMEM((C, OUTW), jnp.float32),
            pltpu.VMEM((C, OUTW * 4), jnp.float32),
            pltpu.VMEM((MAX_TOTAL * 4,), jnp.float32),
            pltpu.VMEM((MAX_TOTAL,), jnp.float32),
            pltpu.VMEM((MAX_TOTAL,), jnp.float32),
            pltpu.SemaphoreType.DMA,
        ],
    )
    def k(osc_hbm, obox_hbm, ob_hbm, oc_hbm, os_hbm,
          csc, cbox, xbox, xcls, xsc, sem):
        wid = lax.axis_index("s") * 2 + lax.axis_index("c")

        @pl.when(wid < B)
        def _():
            b = wid
            iota = lax.iota(jnp.int32, L)
            izero = jnp.zeros((L,), jnp.int32)
            cols1 = jnp.full((L,), 1, jnp.int32)
            cols2 = jnp.full((L,), 2, jnp.int32)
            cols3 = jnp.full((L,), 3, jnp.int32)
            pltpu.sync_copy(osc_hbm.at[pl.ds(b * C, C)], csc)
            pltpu.sync_copy(obox_hbm.at[pl.ds(b * C, C)], cbox)

            def step(s, heads):
                h0, h1 = heads
                cls0 = iota
                cls1 = iota + L
                ok1 = cls1 < C
                v0 = plsc.load_gather(csc, [cls0, h0])
                v1 = plsc.load_gather(csc, [cls1, jnp.where(ok1, h1, 0)],
                                      mask=ok1)
                v1 = jnp.where(ok1, v1, jnp.float32(NEG))
                m = jnp.maximum(jnp.max(v0), jnp.max(v1))
                big = jnp.int32(C)
                j0 = jnp.min(jnp.where(v0 == m, cls0, big))
                j1 = jnp.min(jnp.where(v1 == m, cls1, big))
                j = jnp.minimum(j0, j1)
                hj = jnp.maximum(jnp.max(jnp.where(cls0 == j, h0, -1)),
                                 jnp.max(jnp.where(cls1 == j, h1, -1)))
                valid = m > jnp.float32(NEG / 2)
                jl = jnp.full((L,), j, jnp.int32)
                hl4 = jnp.full((L,), hj, jnp.int32) * 4
                by1 = plsc.load_gather(cbox, [jl, hl4])
                bx1 = plsc.load_gather(cbox, [jl, hl4 + 1])
                by2 = plsc.load_gather(cbox, [jl, hl4 + 2])
                bx2 = plsc.load_gather(cbox, [jl, hl4 + 3])
                vf = jnp.where(valid, jnp.float32(1.0), jnp.float32(0.0))
                lane0 = iota == 0
                sl = jnp.full((L,), s, jnp.int32)
                sl4 = sl * 4
                plsc.store_scatter(xbox, [sl4],
                                   jnp.clip(by1, 0.0, 1.0) * vf, mask=lane0)
                plsc.store_scatter(xbox, [sl4 + 1],
                                   jnp.clip(bx1, 0.0, 1.0) * vf, mask=lane0)
                plsc.store_scatter(xbox, [sl4 + 2],
                                   jnp.clip(by2, 0.0, 1.0) * vf, mask=lane0)
                plsc.store_scatter(xbox, [sl4 + 3],
                                   jnp.clip(bx2, 0.0, 1.0) * vf, mask=lane0)
                plsc.store_scatter(xsc, [sl],
                                   jnp.full((L,), m, jnp.float32) * vf,
                                   mask=lane0)
                plsc.store_scatter(xcls, [sl],
                                   jnp.full((L,), j.astype(jnp.float32)) * vf,
                                   mask=lane0)
                adv = valid
                h0n = h0 + jnp.where((cls0 == j) & adv, 1, 0)
                h1n = h1 + jnp.where((cls1 == j) & adv, 1, 0)
                return h0n, h1n

            lax.fori_loop(0, MAX_TOTAL, step, (izero, izero))
            pltpu.sync_copy(xbox, ob_hbm.at[b])
            pltpu.sync_copy(xcls, oc_hbm.at[b])
            pltpu.sync_copy(xsc, os_hbm.at[b])

    return k(osc, obox)


def kernel(roi_bboxes, pred_deltas, pred_label_probs):
    pad = NP - N
    probs_p = jnp.pad(pred_label_probs, ((0, 0), (0, pad), (0, 0)))
    scores_t = _scores_tc(probs_p)
    d4 = pred_deltas.reshape(B * N * C, 4)
    r3 = roi_bboxes.reshape(B * N, 4)
    planes = (r3[:, 0], r3[:, 1], r3[:, 2], r3[:, 3],
              d4[:, 0], d4[:, 1], d4[:, 2], d4[:, 3])
    osc, obox = _nms_sc(scores_t, planes)
    boxes_flat, cls, sc = _merge_sc(osc, obox)
    return boxes_flat.reshape(B, MAX_TOTAL, 4), cls, sc


# speedup vs baseline: 255.1878x; 4.9117x over previous
import functools

import jax
import jax.numpy as jnp
from jax import lax
from jax.experimental import pallas as pl
from jax.experimental.pallas import tpu as pltpu
from jax.experimental.pallas import tpu_sc as plsc

B, N, C = 4, 20000, 21
NP = 20480
NT = B * C
MAX_TOTAL = 200
SCORE_THR = 0.5
IOU_THR = 0.5
NEG = -1e30
KCAP = 512
OUTW = 256
L = 16
NWORK = 32


def _scores_body(prob_ref, out_ref):
    probs = prob_ref[0]
    m = jnp.max(probs, axis=-1, keepdims=True)
    keep = probs[:, 0:1] < m
    out_ref[0] = jnp.where(keep, probs, 0.0)


def _scores_tc(probs_p):
    blk = 2048
    spec = pl.BlockSpec((1, blk, C), lambda b, i: (b, i, 0))
    return pl.pallas_call(
        _scores_body,
        grid=(B, NP // blk),
        in_specs=[spec],
        out_specs=spec,
        out_shape=jax.ShapeDtypeStruct((B, NP, C), jnp.float32),
    )(probs_p)


def _nms_task(task, sc_hbm, roi_hbm, del_hbm, osc_hbm, obox_hbm,
              dense, widx, wsc, wy1, wx1, wy2, wx2, warea,
              gk0, gk1, gk2, gk3, gk4, gk5, gk6, gk7,
              gy1, gx1, gy2, gx2, gd0, gd1, gd2, gd3,
              ay1, ax1, ay2, ax2, aarea, osc, obox, sem):
    b = lax.shift_right_logical(task * 3121, 16)
    c = task - b * C
    iota = lax.iota(jnp.int32, L)
    fzero = jnp.zeros((L,), jnp.float32)
    izero = jnp.zeros((L,), jnp.int32)
    fneg = jnp.full((L,), NEG, jnp.float32)

    pltpu.sync_copy(sc_hbm.at[b, c], dense)

    def init_body(i, _):
        osc[pl.ds(i * L, L)] = fneg
        return 0
    lax.fori_loop(0, OUTW // L, init_body, 0, unroll=4)

    def initb_body(i, _):
        obox[pl.ds(i * L, L)] = fzero
        return 0
    lax.fori_loop(0, OUTW * 4 // L, initb_body, 0, unroll=4)

    def cnt_body(i, acc):
        v = dense[pl.ds(i * L, L)]
        return acc + jnp.sum((v > SCORE_THR).astype(jnp.int32))
    n05 = lax.fori_loop(0, NP // L, cnt_body, jnp.int32(0), unroll=8)

    def win_pass(t_new, t_lo):
        def body(i, wp):
            v = dense[pl.ds(i * L, L)]
            m = (v > t_new) & (v <= t_lo)
            pos = wp + plsc.cumsum(m.astype(jnp.int32)) - 1
            mm = m & (pos < KCAP)
            plsc.store_scatter(widx, [pos], iota + i * L, mask=mm)
            plsc.store_scatter(wsc, [pos], v, mask=mm)
            return wp + jnp.sum(m.astype(jnp.int32))
        return lax.fori_loop(0, NP // L, body, jnp.int32(0), unroll=4)

    def outer_cond(st):
        accepted, n_rem, t_lo = st
        return (accepted < MAX_TOTAL) & (n_rem > 0)

    def outer_body(st):
        accepted0, n_rem, t_lo = st

        hi0 = jnp.minimum(t_lo, jnp.float32(1.0))
        frac = jnp.max(jnp.full((L,), 0.75 * KCAP, jnp.float32) /
                       jnp.full((L,), jnp.maximum(n_rem, 1).astype(jnp.float32)))
        t0 = jnp.where(n_rem <= KCAP, jnp.float32(SCORE_THR),
                       jnp.maximum(jnp.float32(SCORE_THR),
                                   hi0 - (hi0 - SCORE_THR) * frac))
        cnt0 = win_pass(t0, t_lo)

        def sel_cond(s):
            t, cnt, lo, hi, tries = s
            bad = (cnt > KCAP) | ((cnt == 0) & (t > SCORE_THR))
            return bad & (tries < 30)

        def sel_body(s):
            t, cnt, lo, hi, tries = s
            lo2 = jnp.where(cnt > KCAP, t, lo)
            hi2 = jnp.where(cnt == 0, t, hi)
            t2 = 0.5 * (lo2 + hi2)
            c2 = win_pass(t2, t_lo)
            return t2, c2, lo2, hi2, tries + 1

        t_new, cnt, _, _, _ = lax.while_loop(
            sel_cond, sel_body,
            (t0, cnt0, jnp.float32(SCORE_THR), hi0, jnp.int32(0)))

        def force():
            return jnp.float32(SCORE_THR), win_pass(jnp.float32(SCORE_THR), t_lo)

        t_new, cnt = lax.cond((cnt == 0) & (n_rem > 0), force,
                              lambda: (t_new, cnt))

        nwin = jnp.minimum(cnt, KCAP)
        n_rem2 = jnp.maximum(n_rem - cnt, 0)

        def pad_body(i, _):
            lane = iota + i * L
            m = lane >= nwin
            plsc.store_scatter(wsc, [lane], fneg, mask=m)
            plsc.store_scatter(widx, [lane], izero, mask=m)
            return 0
        lax.fori_loop(0, KCAP // L, pad_body, 0, unroll=4)

        CH = L * 8

        def chunk(k, _):
            @pl.when(k * CH < nwin)
            def _():
                def idx_body(i, _):
                    gi = plsc.load_gather(widx, [iota + k * CH + i * L])
                    ri4 = (gi + b * N) * 4
                    di4 = ((gi + b * N) * C + c) * 4
                    gk0[pl.ds(i * L, L)] = ri4
                    gk1[pl.ds(i * L, L)] = ri4 + 1
                    gk2[pl.ds(i * L, L)] = ri4 + 2
                    gk3[pl.ds(i * L, L)] = ri4 + 3
                    gk4[pl.ds(i * L, L)] = di4
                    gk5[pl.ds(i * L, L)] = di4 + 1
                    gk6[pl.ds(i * L, L)] = di4 + 2
                    gk7[pl.ds(i * L, L)] = di4 + 3
                    return 0
                lax.fori_loop(0, 8, idx_body, 0, unroll=2)
                cps = [
                    pltpu.async_copy(roi_hbm.at[gk0], gy1, sem),
                    pltpu.async_copy(roi_hbm.at[gk1], gx1, sem),
                    pltpu.async_copy(roi_hbm.at[gk2], gy2, sem),
                    pltpu.async_copy(roi_hbm.at[gk3], gx2, sem),
                    pltpu.async_copy(del_hbm.at[gk4], gd0, sem),
                    pltpu.async_copy(del_hbm.at[gk5], gd1, sem),
                    pltpu.async_copy(del_hbm.at[gk6], gd2, sem),
                    pltpu.async_copy(del_hbm.at[gk7], gd3, sem),
                ]
                for cp in cps:
                    cp.wait()

                def dec(i, _):
                    base = k * CH + i * L
                    d0 = gd0[pl.ds(i * L, L)] * 0.1
                    d1 = gd1[pl.ds(i * L, L)] * 0.1
                    d2 = gd2[pl.ds(i * L, L)] * 0.2
                    d3 = gd3[pl.ds(i * L, L)] * 0.2
                    r0 = gy1[pl.ds(i * L, L)]
                    r1 = gx1[pl.ds(i * L, L)]
                    r2 = gy2[pl.ds(i * L, L)]
                    r3 = gx2[pl.ds(i * L, L)]
                    aw = r3 - r1
                    ah = r2 - r0
                    acx = r1 + 0.5 * aw
                    acy = r0 + 0.5 * ah
                    bw = jnp.exp(d3) * aw
                    bh = jnp.exp(d2) * ah
                    bcx = d1 * aw + acx
                    bcy = d0 * ah + acy
                    y1 = bcy - 0.5 * bh
                    x1 = bcx - 0.5 * bw
                    y2 = bh + y1
                    x2 = bw + x1
                    wy1[pl.ds(base, L)] = y1
                    wx1[pl.ds(base, L)] = x1
                    wy2[pl.ds(base, L)] = y2
                    wx2[pl.ds(base, L)] = x2
                    warea[pl.ds(base, L)] = (jnp.maximum(y2 - y1, 0.0) *
                                             jnp.maximum(x2 - x1, 0.0))
                    return 0
                lax.fori_loop(0, 8, dec, 0, unroll=2)
            return 0
        lax.fori_loop(0, KCAP // CH, chunk, 0)

        def window_max():
            def amax_body(i, s):
                bv, bp = s
                v = wsc[pl.ds(i * L, L)]
                p = iota + i * L
                upd = v > bv
                return jnp.where(upd, v, bv), jnp.where(upd, p, bp)
            bv, bp = lax.fori_loop(0, KCAP // L, amax_body,
                                   (fneg, jnp.full((L,), KCAP, jnp.int32)),
                                   unroll=4)
            m = jnp.max(bv)
            j = jnp.min(jnp.where(bv == m, bp, jnp.int32(KCAP)))
            return m, j

        def inner_cond(st3):
            accepted, m, j = st3
            return (accepted < MAX_TOTAL) & (m > jnp.float32(NEG / 2))

        def inner_body(st3):
            accepted, m, j = st3
            jj = jnp.full((L,), j, jnp.int32)
            plsc.store_scatter(wsc, [jj], fneg, mask=iota == 0)
            cy1 = plsc.load_gather(wy1, [jj])
            cx1 = plsc.load_gather(wx1, [jj])
            cy2 = plsc.load_gather(wy2, [jj])
            cx2 = plsc.load_gather(wx2, [jj])
            car = plsc.load_gather(warea, [jj])

            nav = lax.shift_right_logical(accepted + (L - 1), 4)

            def iou_body(i, rej):
                lane_ok = (iota + i * L) < accepted
                yy1 = jnp.maximum(cy1, ay1[pl.ds(i * L, L)])
                xx1 = jnp.maximum(cx1, ax1[pl.ds(i * L, L)])
                yy2 = jnp.minimum(cy2, ay2[pl.ds(i * L, L)])
                xx2 = jnp.minimum(cx2, ax2[pl.ds(i * L, L)])
                inter = jnp.maximum(yy2 - yy1, 0.0) * jnp.maximum(xx2 - xx1, 0.0)
                iou = inter / jnp.maximum(aarea[pl.ds(i * L, L)] + car - inter, 1e-8)
                hit = (iou > IOU_THR) & lane_ok
                return rej | (jnp.max(hit.astype(jnp.int32)) > 0)
            rejected = lax.fori_loop(0, nav, iou_body, jnp.bool_(False))

            @pl.when(jnp.logical_not(rejected))
            def _():
                slot = jnp.full((L,), accepted, jnp.int32)
                lane0 = iota == 0
                plsc.store_scatter(ay1, [slot], cy1, mask=lane0)
                plsc.store_scatter(ax1, [slot], cx1, mask=lane0)
                plsc.store_scatter(ay2, [slot], cy2, mask=lane0)
                plsc.store_scatter(ax2, [slot], cx2, mask=lane0)
                plsc.store_scatter(aarea, [slot], car, mask=lane0)
                plsc.store_scatter(osc, [slot], jnp.full((L,), m, jnp.float32),
                                   mask=lane0)
                slot4 = slot * 4
                plsc.store_scatter(obox, [slot4], cy1, mask=lane0)
                plsc.store_scatter(obox, [slot4 + 1], cx1, mask=lane0)
                plsc.store_scatter(obox, [slot4 + 2], cy2, mask=lane0)
                plsc.store_scatter(obox, [slot4 + 3], cx2, mask=lane0)

            accepted2 = accepted + jnp.where(rejected, 0, 1)
            m2, j2 = window_max()
            return accepted2, m2, j2

        m0, j0 = window_max()
        accepted_f, _, _ = lax.while_loop(inner_cond, inner_body,
                                          (accepted0, m0, j0))
        return accepted_f, n_rem2, t_new

    lax.while_loop(outer_cond, outer_body,
                   (jnp.int32(0), n05, jnp.float32(2.0)))

    pltpu.sync_copy(osc, osc_hbm.at[task])
    pltpu.sync_copy(obox, obox_hbm.at[task])


def _nms_sc(scores_t, roi_flat, del_flat):
    mesh = plsc.VectorSubcoreMesh(core_axis_name="c", subcore_axis_name="s")

    @functools.partial(
        pl.kernel, mesh=mesh,
        compiler_params=pltpu.CompilerParams(needs_layout_passes=False, use_tc_tiling_on_sc=False),
        out_type=[jax.ShapeDtypeStruct((NT, OUTW), jnp.float32),
                  jax.ShapeDtypeStruct((NT, OUTW * 4), jnp.float32)],
        scratch_types=[
            pltpu.VMEM((NP,), jnp.float32),
            pltpu.VMEM((KCAP,), jnp.int32),
            pltpu.VMEM((KCAP,), jnp.float32),
            pltpu.VMEM((KCAP,), jnp.float32),
            pltpu.VMEM((KCAP,), jnp.float32),
            pltpu.VMEM((KCAP,), jnp.float32),
            pltpu.VMEM((KCAP,), jnp.float32),
            pltpu.VMEM((KCAP,), jnp.float32),
            pltpu.VMEM((L * 8,), jnp.int32),
            pltpu.VMEM((L * 8,), jnp.int32),
            pltpu.VMEM((L * 8,), jnp.int32),
            pltpu.VMEM((L * 8,), jnp.int32),
            pltpu.VMEM((L * 8,), jnp.int32),
            pltpu.VMEM((L * 8,), jnp.int32),
            pltpu.VMEM((L * 8,), jnp.int32),
            pltpu.VMEM((L * 8,), jnp.int32),
            pltpu.VMEM((L * 8,), jnp.float32),
            pltpu.VMEM((L * 8,), jnp.float32),
            pltpu.VMEM((L * 8,), jnp.float32),
            pltpu.VMEM((L * 8,), jnp.float32),
            pltpu.VMEM((L * 8,), jnp.float32),
            pltpu.VMEM((L * 8,), jnp.float32),
            pltpu.VMEM((L * 8,), jnp.float32),
            pltpu.VMEM((L * 8,), jnp.float32),
            pltpu.VMEM((OUTW,), jnp.float32),
            pltpu.VMEM((OUTW,), jnp.float32),
            pltpu.VMEM((OUTW,), jnp.float32),
            pltpu.VMEM((OUTW,), jnp.float32),
            pltpu.VMEM((OUTW,), jnp.float32),
            pltpu.VMEM((OUTW,), jnp.float32),
            pltpu.VMEM((OUTW * 4,), jnp.float32),
            pltpu.SemaphoreType.DMA,
        ],
    )
    def k(sc_hbm, roi_hbm, del_hbm, osc_hbm, obox_hbm, *scr):
        wid = lax.axis_index("s") * 2 + lax.axis_index("c")
        for rep in range(3):
            task = wid + rep * NWORK

            @pl.when(task < NT)
            def _():
                _nms_task(task, sc_hbm, roi_hbm, del_hbm,
                          osc_hbm, obox_hbm, *scr)

    return k(scores_t, roi_flat, del_flat)


def _merge_sc(osc, obox):
    mesh = plsc.VectorSubcoreMesh(core_axis_name="c", subcore_axis_name="s")

    @functools.partial(
        pl.kernel, mesh=mesh,
        compiler_params=pltpu.CompilerParams(needs_layout_passes=False, use_tc_tiling_on_sc=False),
        out_type=[jax.ShapeDtypeStruct((B, MAX_TOTAL * 4), jnp.float32),
                  jax.ShapeDtypeStruct((B, MAX_TOTAL), jnp.float32),
                  jax.ShapeDtypeStruct((B, MAX_TOTAL), jnp.float32)],
        scratch_types=[
            pltpu.VMEM((C, OUTW), jnp.float32),
            pltpu.VMEM((C, OUTW * 4), jnp.float32),
            pltpu.VMEM((MAX_TOTAL * 4,), jnp.float32),
            pltpu.VMEM((MAX_TOTAL,), jnp.float32),
            pltpu.VMEM((MAX_TOTAL,), jnp.float32),
            pltpu.SemaphoreType.DMA,
        ],
    )
    def k(osc_hbm, obox_hbm, ob_hbm, oc_hbm, os_hbm,
          csc, cbox, xbox, xcls, xsc, sem):
        wid = lax.axis_index("s") * 2 + lax.axis_index("c")

        @pl.when(wid < B)
        def _():
            b = wid
            iota = lax.iota(jnp.int32, L)
            izero = jnp.zeros((L,), jnp.int32)
            cols1 = jnp.full((L,), 1, jnp.int32)
            cols2 = jnp.full((L,), 2, jnp.int32)
            cols3 = jnp.full((L,), 3, jnp.int32)
            pltpu.sync_copy(osc_hbm.at[pl.ds(b * C, C)], csc)
            pltpu.sync_copy(obox_hbm.at[pl.ds(b * C, C)], cbox)

            def step(s, heads):
                h0, h1 = heads
                cls0 = iota
                cls1 = iota + L
                ok1 = cls1 < C
                v0 = plsc.load_gather(csc, [cls0, h0])
                v1 = plsc.load_gather(csc, [cls1, jnp.where(ok1, h1, 0)],
                                      mask=ok1)
                v1 = jnp.where(ok1, v1, jnp.float32(NEG))
                m = jnp.maximum(jnp.max(v0), jnp.max(v1))
                big = jnp.int32(C)
                j0 = jnp.min(jnp.where(v0 == m, cls0, big))
                j1 = jnp.min(jnp.where(v1 == m, cls1, big))
                j = jnp.minimum(j0, j1)
                hj = jnp.maximum(jnp.max(jnp.where(cls0 == j, h0, -1)),
                                 jnp.max(jnp.where(cls1 == j, h1, -1)))
                valid = m > jnp.float32(NEG / 2)
                jl = jnp.full((L,), j, jnp.int32)
                hl4 = jnp.full((L,), hj, jnp.int32) * 4
                by1 = plsc.load_gather(cbox, [jl, hl4])
                bx1 = plsc.load_gather(cbox, [jl, hl4 + 1])
                by2 = plsc.load_gather(cbox, [jl, hl4 + 2])
                bx2 = plsc.load_gather(cbox, [jl, hl4 + 3])
                vf = jnp.where(valid, jnp.float32(1.0), jnp.float32(0.0))
                lane0 = iota == 0
                sl = jnp.full((L,), s, jnp.int32)
                sl4 = sl * 4
                plsc.store_scatter(xbox, [sl4],
                                   jnp.clip(by1, 0.0, 1.0) * vf, mask=lane0)
                plsc.store_scatter(xbox, [sl4 + 1],
                                   jnp.clip(bx1, 0.0, 1.0) * vf, mask=lane0)
                plsc.store_scatter(xbox, [sl4 + 2],
                                   jnp.clip(by2, 0.0, 1.0) * vf, mask=lane0)
                plsc.store_scatter(xbox, [sl4 + 3],
                                   jnp.clip(bx2, 0.0, 1.0) * vf, mask=lane0)
                plsc.store_scatter(xsc, [sl],
                                   jnp.full((L,), m, jnp.float32) * vf,
                                   mask=lane0)
                plsc.store_scatter(xcls, [sl],
                                   jnp.full((L,), j.astype(jnp.float32)) * vf,
                                   mask=lane0)
                adv = valid
                h0n = h0 + jnp.where((cls0 == j) & adv, 1, 0)
                h1n = h1 + jnp.where((cls1 == j) & adv, 1, 0)
                return h0n, h1n

            lax.fori_loop(0, MAX_TOTAL, step, (izero, izero))
            pltpu.sync_copy(xbox, ob_hbm.at[b])
            pltpu.sync_copy(xcls, oc_hbm.at[b])
            pltpu.sync_copy(xsc, os_hbm.at[b])

    return k(osc, obox)


def kernel(roi_bboxes, pred_deltas, pred_label_probs):
    pad = NP - N
    probs_p = jnp.pad(pred_label_probs, ((0, 0), (0, pad), (0, 0)))
    scores = _scores_tc(probs_p)
    scores_t = jnp.swapaxes(scores, 1, 2)
    roi_flat = roi_bboxes.reshape(B * N * 4)
    del_flat = pred_deltas.reshape(B * N * C * 4)
    osc, obox = _nms_sc(scores_t, roi_flat, del_flat)
    boxes_flat, cls, sc = _merge_sc(osc, obox)
    return boxes_flat.reshape(B, MAX_TOTAL, 4), cls, sc
